# baseline (speedup 1.0000x reference)
import functools

import jax
import jax.numpy as jnp
from jax import lax
from jax.experimental import pallas as pl
from jax.experimental.pallas import tpu as pltpu
from jax.experimental.pallas import tpu_sc as plsc

B, N, IN_C, F_, T, NC = 1024, 30, 128, 64, 16, 4
BB = 32
NWORK = 32
BSC = 384
SPW = BSC // NWORK
BTC = B - BSC

_RSQRT = [float(k) ** -0.5 for k in range(1, 32)]


def _sc_adjacency(graph):
    mesh = plsc.VectorSubcoreMesh(core_axis_name="c", subcore_axis_name="s")

    @functools.partial(
        pl.kernel, mesh=mesh,
        out_type=jax.ShapeDtypeStruct((BSC, 32, 32), jnp.float32),
        scratch_types=[
            pltpu.VMEM((T, N, N), jnp.float32),
            pltpu.VMEM((960,), jnp.float32),
            pltpu.VMEM((32,), jnp.float32),
            pltpu.VMEM((32, 32), jnp.float32),
        ],
    )
    def k(graph_hbm, out_hbm, g_scr, A_buf, dinv_buf, mt_buf):
        wid = lax.axis_index("s") * 2 + lax.axis_index("c")
        iota = lax.iota(jnp.int32, 16)
        zeros16 = jnp.zeros((16,), jnp.float32)

        def body(kk, carry):
            b = wid * SPW + kk
            pltpu.sync_copy(graph_hbm.at[b], g_scr)
            for z in range(4):
                A_buf[pl.ds(896 + 16 * z, 16)] = zeros16
            for r in range(N):
                lo = zeros16
                hi = zeros16
                for t in range(T):
                    lo = lo + g_scr[t, r, pl.ds(0, 16)]
                    hi = hi + g_scr[t, r, pl.ds(N - 16, 16)]
                wlo = jnp.where(lo * (1.0 / T) != 0.0, 1.0, 0.0)
                whi = jnp.where(hi * (1.0 / T) != 0.0, 1.0, 0.0)
                if r <= 13:
                    wlo = wlo + jnp.where(iota == r, 1.0, 0.0)
                else:
                    whi = whi + jnp.where(iota == r - (N - 16), 1.0, 0.0)
                A_buf[pl.ds(r * N, 16)] = wlo
                A_buf[pl.ds(r * N + (N - 16), 16)] = whi
            degv0 = zeros16
            degv1 = zeros16
            for r in range(N):
                degv0 = degv0 + A_buf[pl.ds(r * N, 16)]
                degv1 = degv1 + A_buf[pl.ds(r * N + 16, 16)]
            dv0 = zeros16
            dv1 = zeros16
            for kdeg in range(1, 32):
                fk = float(kdeg)
                dv0 = jnp.where(degv0 == fk, _RSQRT[kdeg - 1], dv0)
                dv1 = jnp.where(degv1 == fk, _RSQRT[kdeg - 1], dv1)
            dinv_buf[pl.ds(0, 16)] = dv0
            dinv_buf[pl.ds(16, 16)] = dv1
            dsh = dinv_buf[pl.ds(N - 16, 16)]
            for r in range(N):
                wlo = A_buf[pl.ds(r * N, 16)]
                whi = A_buf[pl.ds(r * N + (N - 16), 16)]
                mt_buf[r, pl.ds(0, 16)] = wlo * dv0
                mt_buf[r, pl.ds(N - 16, 16)] = whi * dsh
            mt_buf[N, pl.ds(0, 16)] = dv0
            mt_buf[N, pl.ds(16, 16)] = dv1
            pltpu.sync_copy(mt_buf, out_hbm.at[b])
            return carry

        lax.fori_loop(0, SPW, body, 0)

    return k(graph)


def _bmm_t(M, u):
    return lax.dot_general(M, u, (((1,), (1,)), ((0,), (0,))),
                           preferred_element_type=jnp.float32, precision=lax.Precision.HIGHEST)


def _head(M, x, W1, b1, W2, b2, Wlin, blin, WconvT, bconv):
    h = lax.dot_general(x, W1, (((2,), (0,)), ((), ())),
                        preferred_element_type=jnp.float32, precision=lax.Precision.HIGHEST)
    h1 = jnp.maximum(_bmm_t(M, h) + b1[None], 0.0)
    g2 = lax.dot_general(h1, W2, (((2,), (0,)), ((), ())),
                         preferred_element_type=jnp.float32, precision=lax.Precision.HIGHEST)
    h2a = jnp.maximum(_bmm_t(M, g2) + b2[None], 0.0)
    lin = jnp.sum(h2a * Wlin[None], axis=2)
    xl = jnp.maximum(lin + blin, 0.0)
    return jnp.dot(xl, WconvT, preferred_element_type=jnp.float32, precision=lax.Precision.HIGHEST) + bconv


def _tc_mt_body(mt_ref, real_ref, W1_ref, b1_ref, W2_ref, b2_ref,
                Wlin_ref, blin_ref, WconvT_ref, bconv_ref, out_ref):
    payload = mt_ref[...]
    U = payload[:, :N, :N]
    dinv = payload[:, N, :N]
    M = dinv[:, :, None] * U
    out_ref[...] = _head(M, real_ref[...], W1_ref[...], b1_ref[...],
                         W2_ref[...], b2_ref[...], Wlin_ref[...],
                         blin_ref[0, 0], WconvT_ref[...], bconv_ref[...])


def _tc_full_body(graph_ref, real_ref, W1_ref, b1_ref, W2_ref, b2_ref,
                  Wlin_ref, blin_ref, WconvT_ref, bconv_ref, out_ref):
    g = graph_ref[...]
    adj = jnp.sum(g, axis=1) * (1.0 / T)
    w = (adj != 0.0).astype(jnp.float32)
    rr = lax.broadcasted_iota(jnp.int32, (N, N), 0)
    cc = lax.broadcasted_iota(jnp.int32, (N, N), 1)
    eye = (rr == cc).astype(jnp.float32)
    A = w + eye[None]
    deg = jnp.sum(A, axis=1)
    dinv = lax.rsqrt(deg)
    M = dinv[:, :, None] * A * dinv[:, None, :]
    out_ref[...] = _head(M, real_ref[...], W1_ref[...], b1_ref[...],
                         W2_ref[...], b2_ref[...], Wlin_ref[...],
                         blin_ref[0, 0], WconvT_ref[...], bconv_ref[...])


def _weight_specs():
    return [
        pl.BlockSpec((IN_C, F_), lambda i: (0, 0)),
        pl.BlockSpec((1, F_), lambda i: (0, 0)),
        pl.BlockSpec((F_, F_), lambda i: (0, 0)),
        pl.BlockSpec((1, F_), lambda i: (0, 0)),
        pl.BlockSpec((1, F_), lambda i: (0, 0)),
        pl.BlockSpec((1, 1), lambda i: (0, 0)),
        pl.BlockSpec((N, NC), lambda i: (0, 0)),
        pl.BlockSpec((1, NC), lambda i: (0, 0)),
    ]


@jax.jit
def kernel(real, imag, graph, W1, b1, W2, b2, Wlin, blin, Wconv, bconv):
    del imag
    wargs = (W1, b1.reshape(1, F_), W2, b2.reshape(1, F_),
             Wlin.reshape(1, F_), blin.reshape(1, 1), Wconv.T,
             bconv.reshape(1, NC))

    mt = _sc_adjacency(graph)

    off = BSC // BB
    out_tc = pl.pallas_call(
        _tc_full_body,
        grid=(BTC // BB,),
        in_specs=[
            pl.BlockSpec((BB, T, N, N), lambda i: (i + off, 0, 0, 0)),
            pl.BlockSpec((BB, N, IN_C), lambda i: (i + off, 0, 0)),
            *_weight_specs(),
        ],
        out_specs=pl.BlockSpec((BB, NC), lambda i: (i, 0)),
        out_shape=jax.ShapeDtypeStruct((BTC, NC), jnp.float32),
    )(graph, real, *wargs)

    out_sc = pl.pallas_call(
        _tc_mt_body,
        grid=(BSC // BB,),
        in_specs=[
            pl.BlockSpec((BB, 32, 32), lambda i: (i, 0, 0)),
            pl.BlockSpec((BB, N, IN_C), lambda i: (i, 0, 0)),
            *_weight_specs(),
        ],
        out_specs=pl.BlockSpec((BB, NC), lambda i: (i, 0)),
        out_shape=jax.ShapeDtypeStruct((BSC, NC), jnp.float32),
    )(mt, real, *wargs)

    return jnp.concatenate([out_sc, out_tc], axis=0)


# speedup vs baseline: 1.2077x; 1.2077x over previous
import functools

import jax
import jax.numpy as jnp
from jax import lax
from jax.experimental import pallas as pl
from jax.experimental.pallas import tpu as pltpu
from jax.experimental.pallas import tpu_sc as plsc

B, N, IN_C, F_, T, NC = 1024, 30, 128, 64, 16, 4
BB = 32
NWORK = 32
BSC = 384
SPW = BSC // NWORK
BTC = B - BSC

_RSQRT = [float(k) ** -0.5 for k in range(1, 32)]


def _sc_adjacency(graph):
    mesh = plsc.VectorSubcoreMesh(core_axis_name="c", subcore_axis_name="s")

    @functools.partial(
        pl.kernel, mesh=mesh,
        out_type=jax.ShapeDtypeStruct((BSC, 32, 32), jnp.float32),
        scratch_types=[
            pltpu.VMEM((T, N, N), jnp.float32),
            pltpu.VMEM((960,), jnp.float32),
            pltpu.VMEM((32,), jnp.float32),
            pltpu.VMEM((32, 32), jnp.float32),
        ],
    )
    def k(graph_hbm, out_hbm, g_scr, A_buf, dinv_buf, mt_buf):
        wid = lax.axis_index("s") * 2 + lax.axis_index("c")
        iota = lax.iota(jnp.int32, 16)
        zeros16 = jnp.zeros((16,), jnp.float32)

        def body(kk, carry):
            b = wid * SPW + kk
            pltpu.sync_copy(graph_hbm.at[b], g_scr)
            for z in range(4):
                A_buf[pl.ds(896 + 16 * z, 16)] = zeros16
            for r in range(N):
                lo = zeros16
                hi = zeros16
                for t in range(T):
                    lo = lo + g_scr[t, r, pl.ds(0, 16)]
                    hi = hi + g_scr[t, r, pl.ds(N - 16, 16)]
                wlo = jnp.where(lo * (1.0 / T) != 0.0, 1.0, 0.0)
                whi = jnp.where(hi * (1.0 / T) != 0.0, 1.0, 0.0)
                if r <= 13:
                    wlo = wlo + jnp.where(iota == r, 1.0, 0.0)
                else:
                    whi = whi + jnp.where(iota == r - (N - 16), 1.0, 0.0)
                A_buf[pl.ds(r * N, 16)] = wlo
                A_buf[pl.ds(r * N + (N - 16), 16)] = whi
            degv0 = zeros16
            degv1 = zeros16
            for r in range(N):
                degv0 = degv0 + A_buf[pl.ds(r * N, 16)]
                degv1 = degv1 + A_buf[pl.ds(r * N + 16, 16)]
            dv0 = zeros16
            dv1 = zeros16
            for kdeg in range(1, 32):
                fk = float(kdeg)
                dv0 = jnp.where(degv0 == fk, _RSQRT[kdeg - 1], dv0)
                dv1 = jnp.where(degv1 == fk, _RSQRT[kdeg - 1], dv1)
            dinv_buf[pl.ds(0, 16)] = dv0
            dinv_buf[pl.ds(16, 16)] = dv1
            dsh = dinv_buf[pl.ds(N - 16, 16)]
            for r in range(N):
                wlo = A_buf[pl.ds(r * N, 16)]
                whi = A_buf[pl.ds(r * N + (N - 16), 16)]
                mt_buf[r, pl.ds(0, 16)] = wlo * dv0
                mt_buf[r, pl.ds(N - 16, 16)] = whi * dsh
            mt_buf[N, pl.ds(0, 16)] = dv0
            mt_buf[N, pl.ds(16, 16)] = dv1
            pltpu.sync_copy(mt_buf, out_hbm.at[b])
            return carry

        lax.fori_loop(0, SPW, body, 0)

    return k(graph)


def _bmm_t(M, u):
    return lax.dot_general(M, u, (((1,), (1,)), ((0,), (0,))),
                           preferred_element_type=jnp.float32)


def _head(M, x, W1, b1, W2, b2, Wlin, blin, WconvT, bconv):
    h = lax.dot_general(x, W1, (((2,), (0,)), ((), ())),
                        preferred_element_type=jnp.float32)
    h1 = jnp.maximum(_bmm_t(M, h) + b1[None], 0.0)
    g2 = lax.dot_general(h1, W2, (((2,), (0,)), ((), ())),
                         preferred_element_type=jnp.float32)
    h2a = jnp.maximum(_bmm_t(M, g2) + b2[None], 0.0)
    lin = jnp.sum(h2a * Wlin[None], axis=2)
    xl = jnp.maximum(lin + blin, 0.0)
    return jnp.dot(xl, WconvT, preferred_element_type=jnp.float32) + bconv


def _tc_mt_body(mt_ref, real_ref, W1_ref, b1_ref, W2_ref, b2_ref,
                Wlin_ref, blin_ref, WconvT_ref, bconv_ref, out_ref):
    payload = mt_ref[...]
    U = payload[:, :N, :N]
    dinv = payload[:, N, :N]
    M = dinv[:, :, None] * U
    out_ref[...] = _head(M, real_ref[...], W1_ref[...], b1_ref[...],
                         W2_ref[...], b2_ref[...], Wlin_ref[...],
                         blin_ref[0, 0], WconvT_ref[...], bconv_ref[...])


def _tc_full_body(graph_ref, real_ref, W1_ref, b1_ref, W2_ref, b2_ref,
                  Wlin_ref, blin_ref, WconvT_ref, bconv_ref, out_ref):
    g = graph_ref[...]
    adj = jnp.sum(g, axis=1) * (1.0 / T)
    w = (adj != 0.0).astype(jnp.float32)
    rr = lax.broadcasted_iota(jnp.int32, (N, N), 0)
    cc = lax.broadcasted_iota(jnp.int32, (N, N), 1)
    eye = (rr == cc).astype(jnp.float32)
    A = w + eye[None]
    deg = jnp.sum(A, axis=1)
    dinv = lax.rsqrt(deg)
    M = dinv[:, :, None] * A * dinv[:, None, :]
    out_ref[...] = _head(M, real_ref[...], W1_ref[...], b1_ref[...],
                         W2_ref[...], b2_ref[...], Wlin_ref[...],
                         blin_ref[0, 0], WconvT_ref[...], bconv_ref[...])


def _weight_specs():
    return [
        pl.BlockSpec((IN_C, F_), lambda i: (0, 0)),
        pl.BlockSpec((1, F_), lambda i: (0, 0)),
        pl.BlockSpec((F_, F_), lambda i: (0, 0)),
        pl.BlockSpec((1, F_), lambda i: (0, 0)),
        pl.BlockSpec((1, F_), lambda i: (0, 0)),
        pl.BlockSpec((1, 1), lambda i: (0, 0)),
        pl.BlockSpec((N, NC), lambda i: (0, 0)),
        pl.BlockSpec((1, NC), lambda i: (0, 0)),
    ]


@jax.jit
def kernel(real, imag, graph, W1, b1, W2, b2, Wlin, blin, Wconv, bconv):
    del imag
    wargs = (W1, b1.reshape(1, F_), W2, b2.reshape(1, F_),
             Wlin.reshape(1, F_), blin.reshape(1, 1), Wconv.T,
             bconv.reshape(1, NC))

    mt = _sc_adjacency(graph)

    off = BSC // BB
    out_tc = pl.pallas_call(
        _tc_full_body,
        grid=(BTC // BB,),
        in_specs=[
            pl.BlockSpec((BB, T, N, N), lambda i: (i + off, 0, 0, 0)),
            pl.BlockSpec((BB, N, IN_C), lambda i: (i + off, 0, 0)),
            *_weight_specs(),
        ],
        out_specs=pl.BlockSpec((BB, NC), lambda i: (i, 0)),
        out_shape=jax.ShapeDtypeStruct((BTC, NC), jnp.float32),
    )(graph, real, *wargs)

    out_sc = pl.pallas_call(
        _tc_mt_body,
        grid=(BSC // BB,),
        in_specs=[
            pl.BlockSpec((BB, 32, 32), lambda i: (i, 0, 0)),
            pl.BlockSpec((BB, N, IN_C), lambda i: (i, 0, 0)),
            *_weight_specs(),
        ],
        out_specs=pl.BlockSpec((BB, NC), lambda i: (i, 0)),
        out_shape=jax.ShapeDtypeStruct((BSC, NC), jnp.float32),
    )(mt, real, *wargs)

    return jnp.concatenate([out_sc, out_tc], axis=0)
